# baseline (speedup 1.0000x reference)
import functools

import jax
import jax.numpy as jnp
import numpy as np
from jax import lax
from jax.experimental import pallas as pl
from jax.experimental.pallas import tpu as pltpu
from jax.experimental.pallas import tpu_sc as plsc

_B = 16384
_L = 200
_D = 64
_NC = 2
_NS = 16
_NW = _NC * _NS
_BLK = 64
_NBLK = _B // (_NW * _BLK)
_JCH = 2
_LCH = _L // _JCH


_CHUNKS = ((0, 128), (128, 72))


_NSLOT = 16


def _sc_body(s_hbm, c_hbm, semb, cemb, out_hbm, sidx, cidx, rbuf, obuf,
             *sems):
    wid = lax.axis_index("s") * _NC + lax.axis_index("c")

    def _copy(r, q, slot):
        t, j = divmod(q, 2)
        table = semb if t == 0 else cemb
        idx = sidx if t == 0 else cidx
        off, ln = _CHUNKS[j]
        return pltpu.make_async_copy(table.at[idx.at[r, pl.ds(off, ln)]],
                                     rbuf.at[slot, pl.ds(0, ln)], sems[slot])

    def blk_body(blk, carry):
        g = wid * _NBLK + blk
        base = g * _BLK
        pltpu.sync_copy(s_hbm.at[pl.ds(base, _BLK)], sidx)
        pltpu.sync_copy(c_hbm.at[pl.ds(base, _BLK)], cidx)

        def mask_body(r2, mcarry):
            for k in range(13):
                off = _L - 16 if k == 12 else k * 16
                sv = sidx[r2, pl.ds(off, 16)]
                cv = cidx[r2, pl.ds(off, 16)]
                cidx[r2, pl.ds(off, 16)] = jnp.where(sv == 0, 0, cv)
            return mcarry

        lax.fori_loop(0, _BLK, mask_body, 0)

        for cp in range(_NSLOT - 1):
            _copy(cp // 4, cp % 4, cp).start()

        def quad_body(rr, rcarry):
            for part in range(4):
                r = rr * 4 + part
                zero = jnp.zeros((16,), jnp.float32)
                acc = (zero, zero, zero, zero)
                for q in range(4):
                    slot = 4 * part + q
                    _copy(r, q, slot).wait()
                    r_step = (q + 15) // 4
                    q_next = (q + 3) % 4
                    slot_next = (slot + 15) % _NSLOT

                    @pl.when(r + r_step < _BLK)
                    def _():
                        _copy(r + r_step, q_next, slot_next).start()

                    def l_body(l, a, _slot=slot):
                        a0, a1, a2, a3 = a
                        v0 = rbuf[_slot, l, pl.ds(0, 16)]
                        v1 = rbuf[_slot, l, pl.ds(16, 16)]
                        bc = lax.bitcast_convert_type
                        m = jnp.int32(-65536)
                        return (a0 + bc(v0 << 16, jnp.float32),
                                a1 + bc(v1 << 16, jnp.float32),
                                a2 + bc(v0 & m, jnp.float32),
                                a3 + bc(v1 & m, jnp.float32))

                    acc = lax.fori_loop(0, _CHUNKS[q % 2][1], l_body, acc,
                                        unroll=4)
                for d in range(4):
                    obuf[r, pl.ds(d * 16, 16)] = acc[d]
            return rcarry

        lax.fori_loop(0, _BLK // 4, quad_body, 0)
        pltpu.sync_copy(obuf, out_hbm.at[pl.ds(base, _BLK)])
        return carry

    lax.fori_loop(0, _NBLK, blk_body, 0)


@jax.jit
def _sc_gather_sum(s32, c32, semb, cemb):
    return pl.kernel(
        _sc_body,
        mesh=plsc.VectorSubcoreMesh(core_axis_name="c", subcore_axis_name="s"),
        compiler_params=pltpu.CompilerParams(use_tc_tiling_on_sc=False),
        out_type=jax.ShapeDtypeStruct((_B, _D), jnp.float32),
        scratch_types=[
            pltpu.VMEM((_BLK, _L), jnp.int32),
            pltpu.VMEM((_BLK, _L), jnp.int32),
            pltpu.VMEM((_NSLOT, 128, _D // 2), jnp.int32),
            pltpu.VMEM((_BLK, _D), jnp.float32),
        ] + [pltpu.SemaphoreType.DMA] * _NSLOT,
    )(s32, c32, semb, cemb)


def _tc_body(s_ref, pos_ref, sums_ref, w_ref, b_ref, o_ref):
    maskf = (s_ref[...] != 0).astype(jnp.float32)
    denom = jnp.maximum(jnp.sum(maskf, axis=1, keepdims=True), 1.0)
    poss = lax.dot_general(maskf, pos_ref[0:_L, :],
                           (((1,), (0,)), ((), ())),
                           preferred_element_type=jnp.float32)
    h = (sums_ref[...] + poss) / denom
    o_ref[...] = lax.dot_general(h, w_ref[...],
                                 (((1,), (0,)), ((), ())),
                                 preferred_element_type=jnp.float32) + b_ref[...]


_TC_BT = 1024


@jax.jit
def _tc_head(s32, pos_emb, sums, w_p, b_p):
    grid = (_B // _TC_BT,)
    return pl.pallas_call(
        _tc_body,
        grid=grid,
        in_specs=[
            pl.BlockSpec((_TC_BT, _L), lambda i: (i, 0)),
            pl.BlockSpec((256, _D), lambda i: (0, 0)),
            pl.BlockSpec((_TC_BT, _D), lambda i: (i, 0)),
            pl.BlockSpec((_D, 128), lambda i: (0, 0)),
            pl.BlockSpec((1, 128), lambda i: (0, 0)),
        ],
        out_specs=pl.BlockSpec((_TC_BT, 128), lambda i: (i, 0)),
        out_shape=jax.ShapeDtypeStruct((_B, 128), jnp.float32),
    )(s32, pos_emb, sums, w_p, b_p)


def kernel(s_ids, c_ids, shape_emb, color_emb, pos_emb, W, b):
    s32 = s_ids.astype(jnp.int32)
    c32 = c_ids.astype(jnp.int32)

    def _to_packed(t):
        lo = lax.bitcast_convert_type(
            t[:, :_D // 2].astype(jnp.bfloat16), jnp.uint16).astype(jnp.uint32)
        hi = lax.bitcast_convert_type(
            t[:, _D // 2:].astype(jnp.bfloat16), jnp.uint16).astype(jnp.uint32)
        return lax.bitcast_convert_type(lo | (hi << 16), jnp.int32)

    sums = _sc_gather_sum(s32, c32, _to_packed(shape_emb),
                          _to_packed(color_emb))
    w_p = jnp.zeros((_D, 128), jnp.float32).at[:, :2].set(W)
    b_p = jnp.zeros((1, 128), jnp.float32).at[0, :2].set(b)
    outp = _tc_head(s32, pos_emb, sums, w_p, b_p)
    return outp[:, :2]


# speedup vs baseline: 1.0145x; 1.0145x over previous
import functools

import jax
import jax.numpy as jnp
import numpy as np
from jax import lax
from jax.experimental import pallas as pl
from jax.experimental.pallas import tpu as pltpu
from jax.experimental.pallas import tpu_sc as plsc

_B = 16384
_L = 200
_D = 64
_NC = 2
_NS = 16
_NW = _NC * _NS
_BLK = 64
_NBLK = _B // (_NW * _BLK)
_JCH = 2
_LCH = _L // _JCH


_CHUNKS = ((0, 128), (128, 72))


def _sc_body(s_hbm, c_hbm, semb, cemb, out_hbm, sidx, cidx, rbuf, obuf,
             sem0, sem1, sem2, sem3, sem4, sem5, sem6, sem7):
    wid = lax.axis_index("s") * _NC + lax.axis_index("c")
    sems = (sem0, sem1, sem2, sem3, sem4, sem5, sem6, sem7)

    def _copy(r, q, slot):
        t, j = divmod(q, 2)
        table = semb if t == 0 else cemb
        idx = sidx if t == 0 else cidx
        off, ln = _CHUNKS[j]
        return pltpu.make_async_copy(table.at[idx.at[r, pl.ds(off, ln)]],
                                     rbuf.at[slot, pl.ds(0, ln)], sems[slot])

    def blk_body(blk, carry):
        g = wid * _NBLK + blk
        base = g * _BLK
        pltpu.sync_copy(s_hbm.at[pl.ds(base, _BLK)], sidx)
        pltpu.sync_copy(c_hbm.at[pl.ds(base, _BLK)], cidx)

        def mask_body(r2, mcarry):
            for k in range(13):
                off = _L - 16 if k == 12 else k * 16
                sv = sidx[r2, pl.ds(off, 16)]
                cv = cidx[r2, pl.ds(off, 16)]
                cidx[r2, pl.ds(off, 16)] = jnp.where(sv == 0, 0, cv)
            return mcarry

        lax.fori_loop(0, _BLK, mask_body, 0)

        for rp, qp, slotp in ((0, 0, 0), (0, 1, 1), (0, 2, 2), (0, 3, 3),
                              (1, 0, 4), (1, 1, 5), (1, 2, 6)):
            _copy(rp, qp, slotp).start()

        def pair_body(rr, rcarry):
            for half in range(2):
                r = rr * 2 + half
                zero = jnp.zeros((16,), jnp.float32)
                acc = (zero, zero, zero, zero)
                for q in range(4):
                    _copy(r, q, 4 * half + q).wait()
                    r_step = (q + 7) // 4
                    q_next = (q + 3) % 4
                    slot_next = (4 * half + q + 7) % 8

                    @pl.when(r + r_step < _BLK)
                    def _():
                        _copy(r + r_step, q_next, slot_next).start()

                    def l_body(l, a, _q=q, _half=half):
                        a0, a1, a2, a3 = a
                        v0 = rbuf[4 * _half + _q, l, pl.ds(0, 16)]
                        v1 = rbuf[4 * _half + _q, l, pl.ds(16, 16)]
                        bc = lax.bitcast_convert_type
                        m = jnp.int32(-65536)
                        return (a0 + bc(v0 << 16, jnp.float32),
                                a1 + bc(v1 << 16, jnp.float32),
                                a2 + bc(v0 & m, jnp.float32),
                                a3 + bc(v1 & m, jnp.float32))

                    acc = lax.fori_loop(0, _CHUNKS[q % 2][1], l_body, acc,
                                        unroll=4)
                for d in range(4):
                    obuf[r, pl.ds(d * 16, 16)] = acc[d]
            return rcarry

        lax.fori_loop(0, _BLK // 2, pair_body, 0)
        pltpu.sync_copy(obuf, out_hbm.at[pl.ds(base, _BLK)])
        return carry

    lax.fori_loop(0, _NBLK, blk_body, 0)


@jax.jit
def _sc_gather_sum(s32, c32, semb, cemb):
    return pl.kernel(
        _sc_body,
        mesh=plsc.VectorSubcoreMesh(core_axis_name="c", subcore_axis_name="s"),
        compiler_params=pltpu.CompilerParams(use_tc_tiling_on_sc=False),
        out_type=jax.ShapeDtypeStruct((_B, _D), jnp.float32),
        scratch_types=[
            pltpu.VMEM((_BLK, _L), jnp.int32),
            pltpu.VMEM((_BLK, _L), jnp.int32),
            pltpu.VMEM((8, 128, _D // 2), jnp.int32),
            pltpu.VMEM((_BLK, _D), jnp.float32),
        ] + [pltpu.SemaphoreType.DMA] * 8,
    )(s32, c32, semb, cemb)


def _tc_body(s_ref, pos_ref, sums_ref, w_ref, b_ref, o_ref):
    maskf = (s_ref[...] != 0).astype(jnp.float32)
    denom = jnp.maximum(jnp.sum(maskf, axis=1, keepdims=True), 1.0)
    poss = lax.dot_general(maskf, pos_ref[0:_L, :],
                           (((1,), (0,)), ((), ())),
                           preferred_element_type=jnp.float32)
    h = (sums_ref[...] + poss) / denom
    o_ref[...] = lax.dot_general(h, w_ref[...],
                                 (((1,), (0,)), ((), ())),
                                 preferred_element_type=jnp.float32) + b_ref[...]


_TC_BT = 1024


@jax.jit
def _tc_head(s32, pos_emb, sums, w_p, b_p):
    grid = (_B // _TC_BT,)
    return pl.pallas_call(
        _tc_body,
        grid=grid,
        in_specs=[
            pl.BlockSpec((_TC_BT, _L), lambda i: (i, 0)),
            pl.BlockSpec((256, _D), lambda i: (0, 0)),
            pl.BlockSpec((_TC_BT, _D), lambda i: (i, 0)),
            pl.BlockSpec((_D, 128), lambda i: (0, 0)),
            pl.BlockSpec((1, 128), lambda i: (0, 0)),
        ],
        out_specs=pl.BlockSpec((_TC_BT, 128), lambda i: (i, 0)),
        out_shape=jax.ShapeDtypeStruct((_B, 128), jnp.float32),
    )(s32, pos_emb, sums, w_p, b_p)


def kernel(s_ids, c_ids, shape_emb, color_emb, pos_emb, W, b):
    s32 = s_ids.astype(jnp.int32)
    c32 = c_ids.astype(jnp.int32)

    def _to_packed(t):
        lo = lax.bitcast_convert_type(
            t[:, :_D // 2].astype(jnp.bfloat16), jnp.uint16).astype(jnp.uint32)
        hi = lax.bitcast_convert_type(
            t[:, _D // 2:].astype(jnp.bfloat16), jnp.uint16).astype(jnp.uint32)
        return lax.bitcast_convert_type(lo | (hi << 16), jnp.int32)

    sums = _sc_gather_sum(s32, c32, _to_packed(shape_emb),
                          _to_packed(color_emb))
    w_p = jnp.zeros((_D, 128), jnp.float32).at[:, :2].set(W)
    b_p = jnp.zeros((1, 128), jnp.float32).at[0, :2].set(b)
    outp = _tc_head(s32, pos_emb, sums, w_p, b_p)
    return outp[:, :2]
